# baseline (speedup 1.0000x reference)
import math

import jax
import jax.numpy as jnp
from jax.experimental import pallas as pl
from jax.experimental.pallas import tpu as pltpu

KN = 5
EPS = 1e-5
PAD = (KN - 1) // 2


def _conv1_stats_kernel(xp_ref, w1_ref, b1_ref, m_ref, h_ref, stats_ref,
                        *, H, W, C, HWP, B):
    i = pl.program_id(1)

    HW = H * W
    XR2 = (B - 1) * HWP + HW
    XB = XR2 + 2 * PAD * W

    @pl.when(i == 0)
    def _init():
        stats_ref[...] = jnp.zeros_like(stats_ref)

    xflat = xp_ref[...].reshape(B * HWP, C)

    cols = []
    mi = 0
    for kw in range(KN):
        sl = xflat[kw:kw + XB]
        if kw != PAD:
            sl = sl * m_ref[:, mi:mi + 1]
            mi += 1
        cols.append(sl)
    xc = jnp.concatenate(cols, axis=1)

    hs = []
    for j in range(B):
        acc = jnp.zeros((HW, C), jnp.float32)
        for kh in range(KN):
            acc = acc + jnp.dot(xc[j * HWP + kh * W:j * HWP + kh * W + HW],
                                w1_ref[kh * KN * C:(kh * KN + KN) * C],
                                preferred_element_type=jnp.float32)
        hs.append(acc + b1_ref[...])
    for j in range(B):
        h_ref[j] = hs[j].astype(jnp.bfloat16)

    stats_ref[0, 0:1, :] += sum(
        jnp.sum(hj, axis=0, keepdims=True) for hj in hs)
    stats_ref[0, 1:2, :] += sum(
        jnp.sum(hj * hj, axis=0, keepdims=True) for hj in hs)


def _apply_kernel(h_ref, xp_ref, stats_ref, w2_ref, b2_ref, w3_ref, b3_ref,
                  gb_ref, o_ref, *, H, W, B, NHW):
    base = PAD * W + PAD
    HW = H * W
    s = jnp.sum(stats_ref[...], axis=0)
    mean = s[0:1] / NHW
    var = jnp.maximum(s[1:2] / NHW - mean * mean, 0.0)
    scale = gb_ref[0:1] * jax.lax.rsqrt(var + EPS)
    shift = gb_ref[1:2] - mean * scale
    w2f = (w2_ref[...] * scale).astype(jnp.bfloat16)
    b2f = b2_ref[...] + jax.lax.dot_general(
        shift, w2_ref[...], (((1,), (1,)), ((), ())),
        preferred_element_type=jnp.float32)

    h = h_ref[...].reshape(B * HW, h_ref.shape[2])
    a = jax.lax.dot_general(
        h, w2f, (((1,), (1,)), ((), ())),
        preferred_element_type=jnp.float32) + b2f
    a = jnp.maximum(a, 0.0).astype(jnp.bfloat16)
    o = jnp.dot(a, w3_ref[...], preferred_element_type=jnp.float32) + b3_ref[...]
    for j in range(B):
        o_ref[j] = (o[j * HW:(j + 1) * HW]
                    + xp_ref[j, base:base + HW, :].astype(jnp.float32))


def kernel(x, w1, b1, w2, b2, w3, b3, gamma, beta):
    N, C, H, W = x.shape
    HW = H * W
    NHW = N * HW
    C2 = 2 * C
    KK = KN * KN

    pad_top = PAD * W + PAD
    lcm_w8 = W * 8 // math.gcd(W, 8)
    HWP = -(-(HW + 2 * pad_top + 2 * PAD) // lcm_w8) * lcm_w8
    x_pad = jnp.pad(x.reshape(N, C, HW),
                    ((0, 0), (0, 0), (pad_top, HWP - HW - pad_top)))
    xp = jnp.transpose(x_pad, (0, 2, 1)).astype(jnp.bfloat16)

    w1col = jnp.transpose(w1, (2, 3, 1, 0)).reshape(KK * C, C).astype(jnp.bfloat16)
    b1r = b1.reshape(1, C)

    B = 8 if N % 16 == 0 else (2 if N % 4 == 0 else 1)
    XB = (B - 1) * HWP + HW + 2 * PAD * W
    w_co = jnp.arange(XB, dtype=jnp.int32) % W
    mcols = [((w_co >= PAD - kw) & (w_co < W + PAD - kw)).astype(jnp.bfloat16)
             for kw in range(KN) if kw != PAD]
    masks = jnp.stack(mcols, axis=1)

    cores = 2 if (N // B) % 2 == 0 else 1
    steps = N // B // cores
    h_raw, stats = pl.pallas_call(
        lambda *a: _conv1_stats_kernel(*a, H=H, W=W, C=C, HWP=HWP, B=B),
        out_shape=(jax.ShapeDtypeStruct((N, HW, C), jnp.bfloat16),
                   jax.ShapeDtypeStruct((cores, 2, C), jnp.float32)),
        grid=(cores, steps),
        in_specs=[
            pl.BlockSpec((B, HWP, C), lambda c, i: (c * steps + i, 0, 0)),
            pl.BlockSpec((KK * C, C), lambda c, i: (0, 0)),
            pl.BlockSpec((1, C), lambda c, i: (0, 0)),
            pl.BlockSpec((XB, KN - 1), lambda c, i: (0, 0)),
        ],
        out_specs=(
            pl.BlockSpec((B, HW, C), lambda c, i: (c * steps + i, 0, 0)),
            pl.BlockSpec((1, 2, C), lambda c, i: (c, 0, 0)),
        ),
        compiler_params=pltpu.CompilerParams(
            dimension_semantics=("parallel", "arbitrary"),
            vmem_limit_bytes=64 * 1024 * 1024),
    )(xp, w1col, b1r, masks)

    w2r = w2[:, :, 0, 0].astype(jnp.float32)
    b2r = b2.reshape(1, C2)
    w3m = jnp.transpose(w3[:, :, 0, 0], (1, 0)).astype(jnp.bfloat16)
    b3r = b3.reshape(1, C)
    gb = jnp.stack([gamma, beta], axis=0)

    out = pl.pallas_call(
        lambda *a: _apply_kernel(*a, H=H, W=W, B=B, NHW=NHW),
        out_shape=jax.ShapeDtypeStruct((N, HW, C), jnp.float32),
        grid=(N // B,),
        in_specs=[
            pl.BlockSpec((B, HW, C), lambda i: (i, 0, 0)),
            pl.BlockSpec((B, HWP, C), lambda i: (i, 0, 0)),
            pl.BlockSpec((cores, 2, C), lambda i: (0, 0, 0)),
            pl.BlockSpec((C2, C), lambda i: (0, 0)),
            pl.BlockSpec((1, C2), lambda i: (0, 0)),
            pl.BlockSpec((C2, C), lambda i: (0, 0)),
            pl.BlockSpec((1, C), lambda i: (0, 0)),
            pl.BlockSpec((2, C), lambda i: (0, 0)),
        ],
        out_specs=pl.BlockSpec((B, HW, C), lambda i: (i, 0, 0)),
        compiler_params=pltpu.CompilerParams(
            dimension_semantics=("parallel",),
            vmem_limit_bytes=64 * 1024 * 1024),
    )(h_raw, xp, stats, w2r, b2r, w3m, b3r, gb)

    return jnp.transpose(out, (0, 2, 1)).reshape(N, C, H, W)


# speedup vs baseline: 1.0247x; 1.0247x over previous
import math

import jax
import jax.numpy as jnp
from jax.experimental import pallas as pl
from jax.experimental.pallas import tpu as pltpu

KN = 5
EPS = 1e-5
PAD = (KN - 1) // 2


def _conv1_stats_kernel(xp_ref, w1_ref, b1_ref, h_ref, stats_ref,
                        *, H, W, C, HWP, B):
    i = pl.program_id(1)

    HW = H * W
    XR2 = (B - 1) * HWP + HW
    XB = XR2 + 2 * PAD * W

    @pl.when(i == 0)
    def _init():
        stats_ref[...] = jnp.zeros_like(stats_ref)

    xflat = xp_ref[...].reshape(B * HWP, C)

    w_co = jax.lax.broadcasted_iota(jnp.int32, (XB, C), 0) % W
    cols = []
    for kw in range(KN):
        sl = xflat[kw:kw + XB]
        lo, hi = PAD - kw, W + PAD - kw
        if lo > 0:
            sl = jnp.where(w_co >= lo, sl, jnp.bfloat16(0))
        if hi < W:
            sl = jnp.where(w_co < hi, sl, jnp.bfloat16(0))
        cols.append(sl)
    xc = jnp.concatenate(cols, axis=1)

    hs = []
    for j in range(B):
        acc = jnp.zeros((HW, C), jnp.float32)
        for kh in range(KN):
            acc = acc + jnp.dot(xc[j * HWP + kh * W:j * HWP + kh * W + HW],
                                w1_ref[kh * KN * C:(kh * KN + KN) * C],
                                preferred_element_type=jnp.float32)
        hs.append(acc + b1_ref[...])
    for j in range(B):
        h_ref[j] = hs[j].astype(jnp.bfloat16)

    stats_ref[0, 0:1, :] += sum(
        jnp.sum(hj, axis=0, keepdims=True) for hj in hs)
    stats_ref[0, 1:2, :] += sum(
        jnp.sum(hj * hj, axis=0, keepdims=True) for hj in hs)


def _apply_kernel(h_ref, xp_ref, stats_ref, w2_ref, b2_ref, w3_ref, b3_ref,
                  gb_ref, o_ref, *, H, W, B, NHW):
    base = PAD * W + PAD
    HW = H * W
    s = jnp.sum(stats_ref[...], axis=0)
    mean = s[0:1] / NHW
    var = jnp.maximum(s[1:2] / NHW - mean * mean, 0.0)
    scale = gb_ref[0:1] * jax.lax.rsqrt(var + EPS)
    shift = gb_ref[1:2] - mean * scale
    w2f = (w2_ref[...] * scale).astype(jnp.bfloat16)
    b2f = b2_ref[...] + jax.lax.dot_general(
        shift, w2_ref[...], (((1,), (1,)), ((), ())),
        preferred_element_type=jnp.float32)

    h = h_ref[...].reshape(B * HW, h_ref.shape[2])
    a = jax.lax.dot_general(
        h, w2f, (((1,), (1,)), ((), ())),
        preferred_element_type=jnp.float32) + b2f
    a = jnp.maximum(a, 0.0).astype(jnp.bfloat16)
    o = jnp.dot(a, w3_ref[...], preferred_element_type=jnp.float32) + b3_ref[...]
    for j in range(B):
        o_ref[j] = (o[j * HW:(j + 1) * HW]
                    + xp_ref[j, base:base + HW, :].astype(jnp.float32))


def kernel(x, w1, b1, w2, b2, w3, b3, gamma, beta):
    N, C, H, W = x.shape
    HW = H * W
    NHW = N * HW
    C2 = 2 * C
    KK = KN * KN

    pad_top = PAD * W + PAD
    lcm_w8 = W * 8 // math.gcd(W, 8)
    HWP = -(-(HW + 2 * pad_top + 2 * PAD) // lcm_w8) * lcm_w8
    x_pad = jnp.pad(x.reshape(N, C, HW),
                    ((0, 0), (0, 0), (pad_top, HWP - HW - pad_top)))
    xp = jnp.transpose(x_pad, (0, 2, 1)).astype(jnp.bfloat16)

    w1col = jnp.transpose(w1, (2, 3, 1, 0)).reshape(KK * C, C).astype(jnp.bfloat16)
    b1r = b1.reshape(1, C)

    B = 8 if N % 16 == 0 else (2 if N % 4 == 0 else 1)
    cores = 2 if (N // B) % 2 == 0 else 1
    steps = N // B // cores
    h_raw, stats = pl.pallas_call(
        lambda *a: _conv1_stats_kernel(*a, H=H, W=W, C=C, HWP=HWP, B=B),
        out_shape=(jax.ShapeDtypeStruct((N, HW, C), jnp.bfloat16),
                   jax.ShapeDtypeStruct((cores, 2, C), jnp.float32)),
        grid=(cores, steps),
        in_specs=[
            pl.BlockSpec((B, HWP, C), lambda c, i: (c * steps + i, 0, 0)),
            pl.BlockSpec((KK * C, C), lambda c, i: (0, 0)),
            pl.BlockSpec((1, C), lambda c, i: (0, 0)),
        ],
        out_specs=(
            pl.BlockSpec((B, HW, C), lambda c, i: (c * steps + i, 0, 0)),
            pl.BlockSpec((1, 2, C), lambda c, i: (c, 0, 0)),
        ),
        compiler_params=pltpu.CompilerParams(
            dimension_semantics=("parallel", "arbitrary"),
            vmem_limit_bytes=64 * 1024 * 1024),
    )(xp, w1col, b1r)

    w2r = w2[:, :, 0, 0].astype(jnp.float32)
    b2r = b2.reshape(1, C2)
    w3m = jnp.transpose(w3[:, :, 0, 0], (1, 0)).astype(jnp.bfloat16)
    b3r = b3.reshape(1, C)
    gb = jnp.stack([gamma, beta], axis=0)

    out = pl.pallas_call(
        lambda *a: _apply_kernel(*a, H=H, W=W, B=B, NHW=NHW),
        out_shape=jax.ShapeDtypeStruct((N, HW, C), jnp.float32),
        grid=(N // B,),
        in_specs=[
            pl.BlockSpec((B, HW, C), lambda i: (i, 0, 0)),
            pl.BlockSpec((B, HWP, C), lambda i: (i, 0, 0)),
            pl.BlockSpec((cores, 2, C), lambda i: (0, 0, 0)),
            pl.BlockSpec((C2, C), lambda i: (0, 0)),
            pl.BlockSpec((1, C2), lambda i: (0, 0)),
            pl.BlockSpec((C2, C), lambda i: (0, 0)),
            pl.BlockSpec((1, C), lambda i: (0, 0)),
            pl.BlockSpec((2, C), lambda i: (0, 0)),
        ],
        out_specs=pl.BlockSpec((B, HW, C), lambda i: (i, 0, 0)),
        compiler_params=pltpu.CompilerParams(
            dimension_semantics=("parallel",),
            vmem_limit_bytes=64 * 1024 * 1024),
    )(h_raw, xp, stats, w2r, b2r, w3m, b3r, gb)

    return jnp.transpose(out, (0, 2, 1)).reshape(N, C, H, W)
